# baseline (speedup 1.0000x reference)
import functools

import jax
import jax.numpy as jnp
from jax import lax
from jax.experimental import pallas as pl
from jax.experimental.pallas import tpu as pltpu
from jax.experimental.pallas import tpu_sc as plsc

VOCAB = 100000
EMBED = 128
BATCH = 4096
HIST = 50
HPAD = 56

NW = 32
SPC = 2
NSPLIT = 2
NSEQ = BATCH // NSPLIT
SEQ_W = NSEQ // NW
NCHUNK = SEQ_W // SPC


@functools.partial(
    pl.kernel,
    mesh=plsc.VectorSubcoreMesh(core_axis_name="c", subcore_axis_name="s"),
    out_type=jax.ShapeDtypeStruct((NSEQ, HIST, EMBED), jnp.float32),
    scratch_types=[
        pltpu.VMEM((SEQ_W, HPAD), jnp.int32),
        pltpu.VMEM((SPC, HIST, EMBED), jnp.float32),
        pltpu.VMEM((SPC, HIST, EMBED), jnp.float32),
        pltpu.VMEM((SPC, HIST, EMBED), jnp.float32),
        pltpu.VMEM((SPC, HIST, EMBED), jnp.float32),
        pltpu.SemaphoreType.DMA,
        pltpu.SemaphoreType.DMA,
        pltpu.SemaphoreType.DMA,
        pltpu.SemaphoreType.DMA,
        pltpu.SemaphoreType.DMA,
        pltpu.SemaphoreType.DMA,
        pltpu.SemaphoreType.DMA,
        pltpu.SemaphoreType.DMA,
    ],
)
def _embed_gather(idx_hbm, table_hbm, out_hbm, idx_v, r0, r1, r2, r3,
                  g0, g1, g2, g3, w0, w1, w2, w3):
    wid = lax.axis_index("s") * 2 + lax.axis_index("c")
    seq0 = wid * SEQ_W
    bufs = (r0, r1, r2, r3)
    gsems = (g0, g1, g2, g3)
    wsems = (w0, w1, w2, w3)
    pltpu.sync_copy(idx_hbm.at[wid], idx_v)

    def fire_gathers(c, b):
        for sub in range(SPC):
            pltpu.async_copy(
                table_hbm.at[idx_v.at[c * SPC + sub, pl.ds(0, HIST)]],
                bufs[b].at[sub], gsems[b])

    def wait_gathers(c, b):
        for sub in range(SPC):
            pltpu.make_async_copy(
                table_hbm.at[idx_v.at[c * SPC + sub, pl.ds(0, HIST)]],
                bufs[b].at[sub], gsems[b]).wait()

    def wb_dst(c):
        return out_hbm.at[pl.ds(seq0 + c * SPC, SPC)]

    fire_gathers(0, 0)
    fire_gathers(1, 1)

    def body(i, carry):
        for b in range(4):
            s = i * 4 + b
            nb = (b + 2) % 4
            wait_gathers(s, b)
            pltpu.async_copy(bufs[b], wb_dst(s), wsems[b])

            @pl.when(s >= 2)
            def _():
                pltpu.make_async_copy(bufs[nb], wb_dst(s - 2),
                                      wsems[nb]).wait()

            fire_gathers(s + 2, nb)
        return carry

    lax.fori_loop(0, (NCHUNK - 4) // 4, body, 0)

    for s in range(NCHUNK - 4, NCHUNK):
        b = s % 4
        nb = (b + 2) % 4
        wait_gathers(s, b)
        pltpu.async_copy(bufs[b], wb_dst(s), wsems[b])
        pltpu.make_async_copy(bufs[nb], wb_dst(s - 2), wsems[nb]).wait()
        if s + 2 < NCHUNK:
            fire_gathers(s + 2, nb)

    for s in (NCHUNK - 2, NCHUNK - 1):
        b = s % 4
        pltpu.make_async_copy(bufs[b], wb_dst(s), wsems[b]).wait()


def kernel(input_seqs, table):
    idx = input_seqs.astype(jnp.int32)
    idx = jnp.pad(idx, ((0, 0), (0, HPAD - HIST)))
    parts = []
    for p in range(NSPLIT):
        part = idx[p * NSEQ:(p + 1) * NSEQ].reshape(NW, SEQ_W, HPAD)
        parts.append(_embed_gather(part, table))
    return jnp.concatenate(parts, axis=0)


# speedup vs baseline: 2.8422x; 2.8422x over previous
import functools

import jax
import jax.numpy as jnp
from jax import lax
from jax.experimental import pallas as pl
from jax.experimental.pallas import tpu as pltpu
from jax.experimental.pallas import tpu_sc as plsc

VOCAB = 100000
EMBED = 128
BATCH = 4096
HIST = 50

NTOT = BATCH * HIST
NW = 32
PER_W = NTOT // NW
CHUNK = 128
NCHUNK = PER_W // CHUNK


@functools.partial(
    pl.kernel,
    mesh=plsc.VectorSubcoreMesh(core_axis_name="c", subcore_axis_name="s"),
    out_type=jax.ShapeDtypeStruct((NTOT, EMBED), jnp.float32),
    scratch_types=[
        pltpu.VMEM((NCHUNK, CHUNK), jnp.int32),
        pltpu.VMEM((CHUNK, EMBED), jnp.float32),
        pltpu.VMEM((CHUNK, EMBED), jnp.float32),
        pltpu.VMEM((CHUNK, EMBED), jnp.float32),
        pltpu.VMEM((CHUNK, EMBED), jnp.float32),
        pltpu.SemaphoreType.DMA,
        pltpu.SemaphoreType.DMA,
        pltpu.SemaphoreType.DMA,
        pltpu.SemaphoreType.DMA,
        pltpu.SemaphoreType.DMA,
        pltpu.SemaphoreType.DMA,
        pltpu.SemaphoreType.DMA,
        pltpu.SemaphoreType.DMA,
    ],
)
def _embed_gather(idx_hbm, table_hbm, out_hbm, idx_v, r0, r1, r2, r3,
                  g0, g1, g2, g3, w0, w1, w2, w3):
    wid = lax.axis_index("s") * 2 + lax.axis_index("c")
    base = wid * PER_W
    bufs = (r0, r1, r2, r3)
    gsems = (g0, g1, g2, g3)
    wsems = (w0, w1, w2, w3)
    pltpu.sync_copy(idx_hbm.at[wid], idx_v)

    def wb_dst(c):
        return out_hbm.at[pl.ds(base + c * CHUNK, CHUNK)]

    pltpu.async_copy(table_hbm.at[idx_v.at[0]], r0, g0)
    pltpu.async_copy(table_hbm.at[idx_v.at[1]], r1, g1)

    def body(i, carry):
        for b in range(4):
            s = i * 4 + b
            nb = (b + 2) % 4
            pltpu.make_async_copy(table_hbm.at[idx_v.at[s]], bufs[b],
                                  gsems[b]).wait()
            pltpu.async_copy(bufs[b], wb_dst(s), wsems[b])

            @pl.when(s >= 2)
            def _():
                pltpu.make_async_copy(bufs[nb], wb_dst(s - 2),
                                      wsems[nb]).wait()

            pltpu.async_copy(table_hbm.at[idx_v.at[s + 2]], bufs[nb],
                             gsems[nb])
        return carry

    lax.fori_loop(0, (NCHUNK - 2) // 4, body, 0)

    for s in (NCHUNK - 2, NCHUNK - 1):
        b = s % 4
        nb = (b + 2) % 4
        pltpu.make_async_copy(table_hbm.at[idx_v.at[s]], bufs[b],
                              gsems[b]).wait()
        pltpu.async_copy(bufs[b], wb_dst(s), wsems[b])
        pltpu.make_async_copy(bufs[nb], wb_dst(s - 2), wsems[nb]).wait()

    for s in (NCHUNK - 2, NCHUNK - 1):
        b = s % 4
        pltpu.make_async_copy(bufs[b], wb_dst(s), wsems[b]).wait()


def kernel(input_seqs, table):
    idx = input_seqs.astype(jnp.int32).T.reshape(NW, NCHUNK, CHUNK)
    out = _embed_gather(idx, table)
    return out.reshape(HIST, BATCH, EMBED).transpose(1, 0, 2)
